# baseline (speedup 1.0000x reference)
import jax
import jax.numpy as jnp
from jax.experimental import pallas as pl
from jax.experimental.pallas import tpu as pltpu

_CF = 32
_SPC = _CF // 4
_NBUF = 3


def kernel(frames):
    B, T, H, W = frames.shape
    Ts = T // 4
    cpb = T // _CF
    NB = B * cpb

    def body(in_hbm, slow_hbm, fast_hbm, buf, sem_in, sem_out):
        d_in = {}
        pend_out = {}

        def start_in(k):
            if k - _NBUF >= 0:
                for d in pend_out.pop(k - _NBUF):
                    d.wait()
            b, q = divmod(k, cpb)
            slot = k % _NBUF
            d = pltpu.make_async_copy(
                in_hbm.at[b, pl.ds(q * _CF, _CF)], buf.at[slot],
                sem_in.at[slot],
            )
            d.start()
            d_in[k] = d

        for k in range(min(_NBUF - 1, NB)):
            start_in(k)

        for g in range(NB):
            if g + _NBUF - 1 < NB:
                start_in(g + _NBUF - 1)
            d_in.pop(g).wait()
            b, q = divmod(g, cpb)
            slot = g % _NBUF
            outs = []
            d = pltpu.make_async_copy(
                buf.at[slot], fast_hbm.at[b, pl.ds(q * _CF, _CF)],
                sem_out.at[slot],
            )
            d.start()
            outs.append(d)
            for j in range(_SPC):
                p = _SPC * q + j
                o = (21 * p) // 5 - _CF * q
                d = pltpu.make_async_copy(
                    buf.at[slot, pl.ds(o, 1)], slow_hbm.at[b, pl.ds(p, 1)],
                    sem_out.at[slot],
                )
                d.start()
                outs.append(d)
            pend_out[g] = outs

        for k in sorted(pend_out):
            for d in pend_out[k]:
                d.wait()

    slow, fast = pl.pallas_call(
        body,
        in_specs=[pl.BlockSpec(memory_space=pl.ANY)],
        out_specs=(
            pl.BlockSpec(memory_space=pl.ANY),
            pl.BlockSpec(memory_space=pl.ANY),
        ),
        out_shape=(
            jax.ShapeDtypeStruct((B, Ts, H, W), frames.dtype),
            jax.ShapeDtypeStruct((B, T, H, W), frames.dtype),
        ),
        scratch_shapes=[
            pltpu.VMEM((_NBUF, _CF, H, W), frames.dtype),
            pltpu.SemaphoreType.DMA((_NBUF,)),
            pltpu.SemaphoreType.DMA((_NBUF,)),
        ],
    )(frames)
    return (slow, fast)


# speedup vs baseline: 1.0013x; 1.0013x over previous
import jax
import jax.numpy as jnp
from jax.experimental import pallas as pl
from jax.experimental.pallas import tpu as pltpu

_CF = 16
_SPC = _CF // 4
_NBUF = 6


def kernel(frames):
    B, T, H, W = frames.shape
    Ts = T // 4
    cpb = T // _CF
    NB = B * cpb

    def body(in_hbm, slow_hbm, fast_hbm, buf, sem_in, sem_out):
        d_in = {}
        pend_out = {}

        def start_in(k):
            if k - _NBUF >= 0:
                for d in pend_out.pop(k - _NBUF):
                    d.wait()
            b, q = divmod(k, cpb)
            slot = k % _NBUF
            d = pltpu.make_async_copy(
                in_hbm.at[b, pl.ds(q * _CF, _CF)], buf.at[slot],
                sem_in.at[slot],
            )
            d.start()
            d_in[k] = d

        for k in range(min(_NBUF - 1, NB)):
            start_in(k)

        for g in range(NB):
            if g + _NBUF - 1 < NB:
                start_in(g + _NBUF - 1)
            d_in.pop(g).wait()
            b, q = divmod(g, cpb)
            slot = g % _NBUF
            outs = []
            d = pltpu.make_async_copy(
                buf.at[slot], fast_hbm.at[b, pl.ds(q * _CF, _CF)],
                sem_out.at[slot],
            )
            d.start()
            outs.append(d)
            for j in range(_SPC):
                p = _SPC * q + j
                o = (21 * p) // 5 - _CF * q
                d = pltpu.make_async_copy(
                    buf.at[slot, pl.ds(o, 1)], slow_hbm.at[b, pl.ds(p, 1)],
                    sem_out.at[slot],
                )
                d.start()
                outs.append(d)
            pend_out[g] = outs

        for k in sorted(pend_out):
            for d in pend_out[k]:
                d.wait()

    slow, fast = pl.pallas_call(
        body,
        in_specs=[pl.BlockSpec(memory_space=pl.ANY)],
        out_specs=(
            pl.BlockSpec(memory_space=pl.ANY),
            pl.BlockSpec(memory_space=pl.ANY),
        ),
        out_shape=(
            jax.ShapeDtypeStruct((B, Ts, H, W), frames.dtype),
            jax.ShapeDtypeStruct((B, T, H, W), frames.dtype),
        ),
        scratch_shapes=[
            pltpu.VMEM((_NBUF, _CF, H, W), frames.dtype),
            pltpu.SemaphoreType.DMA((_NBUF,)),
            pltpu.SemaphoreType.DMA((_NBUF,)),
        ],
    )(frames)
    return (slow, fast)
